# baseline (speedup 1.0000x reference)
import functools

import jax
import jax.numpy as jnp
from jax import lax
from jax.experimental import pallas as pl
from jax.experimental.pallas import tpu as pltpu
from jax.experimental.pallas import tpu_sc as plsc

G = 64
G3 = G * G * G
SPACING = 2.0
IMG_W = 640
IMG_H = 512
HW = (IMG_H // 2) * (IMG_W // 2)

B = 2
C = 8
J = 16

IDX_ROWS = G3 // 128
BLK_ROWS = 256
N_CHUNKS = IDX_ROWS // BLK_ROWS

NW = 32
P_TOT = B * G3
RPW = P_TOT // NW
SUP = 1024
SUB = 128
SUBROWS = SUB // 128
SUPROWS = SUP // 128
NSUB = SUP // SUB
NSUP = RPW // SUP


def _idx_body(coef_ref, idx_ref):
    bc = pl.program_id(0)
    ch = pl.program_id(1)
    row = lax.broadcasted_iota(jnp.int32, (BLK_ROWS, 128), 0)
    col = lax.broadcasted_iota(jnp.int32, (BLK_ROWS, 128), 1)
    p = ch * (BLK_ROWS * 128) + row * 128 + col
    gi = p >> 12
    gj = (p >> 6) & 63
    gk = p & 63

    def cf(k):
        return coef_ref[bc, k]

    fi = (gi.astype(jnp.float32) - 32.0) * SPACING
    fj = (gj.astype(jnp.float32) - 32.0) * SPACING
    fk = (gk.astype(jnp.float32) - 32.0) * SPACING

    def bf(x):
        return x.astype(jnp.bfloat16).astype(jnp.float32)

    px = bf(fi + cf(12))
    py = bf(fj + cf(13))
    pz = bf(fk + cf(14))

    p0 = ((px * bf(cf(0)) + py * bf(cf(3))) + pz * bf(cf(6))) + bf(cf(9))
    p1 = ((px * bf(cf(1)) + py * bf(cf(4))) + pz * bf(cf(7))) + bf(cf(10))
    p2 = ((px * bf(cf(2)) + py * bf(cf(5))) + pz * bf(cf(8))) + bf(cf(11))

    u = p0 / p2
    v = p1 / p2
    fx = cf(15)
    fy = cf(16)
    cx = cf(17)
    cy = cf(18)
    k1 = cf(19)
    k2 = cf(20)
    un = (u - cx) / fx
    vn = (v - cy) / fy
    r2 = un * un + vn * vn
    dist = 1.0 + k1 * r2 + k2 * r2 * r2
    ud = un * dist * fx + cx
    vd = vn * dist * fy + cy
    ud = jnp.clip(ud, 0.0, float(IMG_W - 1))
    vd = jnp.clip(vd, 0.0, float(IMG_H - 1))
    idx_ref[0] = (vd / 2.0).astype(jnp.int32) * (IMG_W // 2) + (ud / 2.0).astype(jnp.int32)


def _compute_idx(coef):
    return pl.pallas_call(
        _idx_body,
        grid=(B * C, N_CHUNKS),
        in_specs=[
            pl.BlockSpec((B * C, 24), lambda i, j: (0, 0), memory_space=pltpu.SMEM),
        ],
        out_specs=pl.BlockSpec((1, BLK_ROWS, 128), lambda i, j: (i, j, 0)),
        out_shape=jax.ShapeDtypeStruct((B * C, IDX_ROWS, 128), jnp.int32),
    )(coef)


def _tr_table_body(src_ref, dst_ref):
    dst_ref[0] = src_ref[0].T


def _transpose_table(hm3):
    return pl.pallas_call(
        _tr_table_body,
        grid=(B * C, (IMG_H // 2) * (IMG_W // 2) // 512),
        in_specs=[pl.BlockSpec((1, J, 512), lambda i, j: (i, 0, j))],
        out_specs=pl.BlockSpec((1, 512, J), lambda i, j: (i, j, 0)),
        out_shape=jax.ShapeDtypeStruct((B * C, HW, J), jnp.float32),
    )(hm3)


def _transpose_out(outp3):
    return pl.pallas_call(
        _tr_table_body,
        grid=(B, G3 // 512),
        in_specs=[pl.BlockSpec((1, 512, J), lambda i, j: (i, j, 0))],
        out_specs=pl.BlockSpec((1, J, 512), lambda i, j: (i, 0, j)),
        out_shape=jax.ShapeDtypeStruct((B, J, G3), jnp.float32),
    )(outp3)


def _sc_body(idx_hbm, table_hbm, out_hbm, idx_v, rows_v, urows, outbuf, isem, gsem, usem):
    w = lax.axis_index("s") * 2 + lax.axis_index("c")

    def idx_src(si, c):
        row0 = w * RPW + si * SUP
        b = row0 // G3
        pnt0 = row0 - b * G3
        base = pl.multiple_of(((b * C + c) * G3 + pnt0) // 128, SUPROWS)
        return idx_hbm.at[pl.ds(base, SUPROWS)]

    for c in range(C):
        pltpu.async_copy(idx_src(0, c), idx_v.at[0, c], isem)

    def sup_body(si, carry):
        row0 = pl.multiple_of(w * RPW + si * SUP, SUP)
        slot = lax.rem(si, 2)
        nslot = 1 - slot
        for c in range(C):
            pltpu.make_async_copy(idx_src(si, c), idx_v.at[slot, c], isem).wait()

        @pl.when(si + 1 < NSUP)
        def _():
            for c in range(C):
                pltpu.async_copy(idx_src(si + 1, c), idx_v.at[nslot, c], isem)

        b = row0 // G3
        lanes = lax.iota(jnp.int32, 16)
        unis = []
        udescs = []
        for c in range(C):
            v0 = idx_v[slot, c, 0, pl.ds(0, 16)]

            def chk(t, mv):
                v = idx_v[slot, c, t >> 3, pl.ds((t & 7) * 16, 16)]
                return (jnp.minimum(mv[0], v), jnp.maximum(mv[1], v))

            mnv, mxv = lax.fori_loop(1, SUP // 16, chk, (v0, v0))
            mn = jnp.min(mnv)
            mx = jnp.max(mxv)
            unis.append((mn == mx) & (mn < 0))
            jidx = ((b * C + c) * J + lanes) * HW + mn
            udescs.append(pltpu.async_copy(table_hbm.at[jidx], urows.at[c, 0], usem))
        for d in udescs:
            d.wait()
        all_uni = unis[0]
        for c in range(1, C):
            all_uni = all_uni & unis[c]

        @pl.when(all_uni)
        def _():
            acc = urows[0, 0]
            for c in range(1, C):
                acc = acc + urows[c, 0]
            accv = acc * (1.0 / C)
            for j in range(J):
                sj = jnp.max(jnp.where(lanes == j, accv, jnp.float32(-3.4e38)))
                splat = jnp.full((16,), sj, jnp.float32)

                @plsc.parallel_loop(0, SUP // 16, unroll=8)
                def _(t, j=j, splat=splat):
                    outbuf[j, pl.ds(t * 16, 16)] = splat

        @pl.when(jnp.logical_not(all_uni))
        def _():
            for c in range(C):
                @pl.when(unis[c])
                def _(c=c):
                    row = urows[c, 0]

                    @plsc.parallel_loop(0, J * (SUB // 16), unroll=4)
                    def _(t):
                        j = t >> 3
                        g = t & 7
                        sj = jnp.max(jnp.where(lanes == j, row, jnp.float32(-3.4e38)))
                        rows_v[c, j, pl.ds(g * 16, 16)] = jnp.full((16,), sj, jnp.float32)

            for s in range(NSUB):
                for c in range(C):
                    @pl.when(jnp.logical_not(unis[c]))
                    def _(c=c, s=s):
                        def gather_j(j, _):
                            base = ((b * C + c) * J + j) * HW
                            descs = [
                                pltpu.async_copy(
                                    table_hbm.at[
                                        idx_v[slot, c, s, pl.ds(g * 16, 16)] + base
                                    ],
                                    rows_v.at[c, j, pl.ds(g * 16, 16)],
                                    gsem,
                                )
                                for g in range(8)
                            ]
                            for d in descs:
                                d.wait()
                            return 0

                        lax.fori_loop(0, J, gather_j, 0)

                @plsc.parallel_loop(0, J * (SUB // 16), unroll=4)
                def _(t, s=s):
                    j = t >> 3
                    g = t & 7
                    acc = rows_v[0, j, pl.ds(g * 16, 16)]
                    for c in range(1, C):
                        acc = acc + rows_v[c, j, pl.ds(g * 16, 16)]
                    outbuf[j, pl.ds(s * SUB + g * 16, 16)] = acc * (1.0 / C)

        b_out = row0 // G3
        pnt0_out = row0 - b_out * G3
        pltpu.sync_copy(outbuf, out_hbm.at[b_out, :, pl.ds(pnt0_out, SUP)])
        return carry

    lax.fori_loop(0, NSUP, sup_body, 0)


@functools.cache
def _sc_gather():
    return pl.kernel(
        _sc_body,
        out_type=jax.ShapeDtypeStruct((B, J, G3), jnp.float32),
        mesh=plsc.VectorSubcoreMesh(
            core_axis_name="c", subcore_axis_name="s", num_cores=2, num_subcores=16
        ),
        scratch_types=[
            pltpu.VMEM((2, C, SUPROWS, 128), jnp.int32),
            pltpu.VMEM((C, J, SUB), jnp.float32),
            pltpu.VMEM((C, 1, J), jnp.float32),
            pltpu.VMEM((J, SUP), jnp.float32),
            pltpu.SemaphoreType.DMA,
            pltpu.SemaphoreType.DMA,
            pltpu.SemaphoreType.DMA,
        ],
        compiler_params=pltpu.CompilerParams(
            use_tc_tiling_on_sc=False,
            disable_bounds_checks=True,
            disable_semaphore_checks=True,
            needs_layout_passes=False,
        ),
    )


def kernel(heatmaps, center, cameraMatrices, intrinsicMatrices, distortionCoefficients):
    Bv, Cv, Jv, H, W = heatmaps.shape
    Mf = cameraMatrices.reshape(B * C, 12)
    cen = jnp.broadcast_to(center[:, None, :], (B, C, 3)).reshape(B * C, 3)
    fx = intrinsicMatrices[:, :, 0, 0].reshape(-1, 1)
    fy = intrinsicMatrices[:, :, 1, 1].reshape(-1, 1)
    cx = intrinsicMatrices[:, :, 2, 0].reshape(-1, 1)
    cy = intrinsicMatrices[:, :, 2, 1].reshape(-1, 1)
    k1 = distortionCoefficients[:, :, 0, 0].reshape(-1, 1)
    k2 = distortionCoefficients[:, :, 0, 1].reshape(-1, 1)
    pad = jnp.zeros((B * C, 3), jnp.float32)
    coef = jnp.concatenate([Mf, cen, fx, fy, cx, cy, k1, k2, pad], axis=1)

    idx = _compute_idx(coef)
    idx2d = idx.reshape(B * C * IDX_ROWS, 128)
    table = heatmaps.reshape(B * C * J * H * W)
    outp = _sc_gather()(idx2d, table)
    return outp.reshape(B, J, G, G, G)


# speedup vs baseline: 52.1368x; 52.1368x over previous
import functools

import jax
import jax.numpy as jnp
from jax import lax
from jax.experimental import pallas as pl
from jax.experimental.pallas import tpu as pltpu
from jax.experimental.pallas import tpu_sc as plsc

G = 64
G3 = G * G * G
SPACING = 2.0
IMG_W = 640
IMG_H = 512
HW = (IMG_H // 2) * (IMG_W // 2)

B = 2
C = 8
J = 16

IDX_ROWS = G3 // 128
BLK_ROWS = 256
N_CHUNKS = IDX_ROWS // BLK_ROWS

NW = 32
P_TOT = B * G3
RPW = P_TOT // NW
SUP = 1024
SUB = 128
SUBROWS = SUB // 128
SUPROWS = SUP // 128
NSUB = SUP // SUB
NSUP = RPW // SUP


def _idx_body(coef_ref, idx_ref):
    bc = pl.program_id(0)
    ch = pl.program_id(1)
    row = lax.broadcasted_iota(jnp.int32, (BLK_ROWS, 128), 0)
    col = lax.broadcasted_iota(jnp.int32, (BLK_ROWS, 128), 1)
    p = ch * (BLK_ROWS * 128) + row * 128 + col
    gi = p >> 12
    gj = (p >> 6) & 63
    gk = p & 63

    def cf(k):
        return coef_ref[bc, k]

    fi = (gi.astype(jnp.float32) - 32.0) * SPACING
    fj = (gj.astype(jnp.float32) - 32.0) * SPACING
    fk = (gk.astype(jnp.float32) - 32.0) * SPACING

    def bf(x):
        return x.astype(jnp.bfloat16).astype(jnp.float32)

    px = bf(fi + cf(12))
    py = bf(fj + cf(13))
    pz = bf(fk + cf(14))

    p0 = ((px * bf(cf(0)) + py * bf(cf(3))) + pz * bf(cf(6))) + bf(cf(9))
    p1 = ((px * bf(cf(1)) + py * bf(cf(4))) + pz * bf(cf(7))) + bf(cf(10))
    p2 = ((px * bf(cf(2)) + py * bf(cf(5))) + pz * bf(cf(8))) + bf(cf(11))

    u = p0 / p2
    v = p1 / p2
    fx = cf(15)
    fy = cf(16)
    cx = cf(17)
    cy = cf(18)
    k1 = cf(19)
    k2 = cf(20)
    un = (u - cx) / fx
    vn = (v - cy) / fy
    r2 = un * un + vn * vn
    dist = 1.0 + k1 * r2 + k2 * r2 * r2
    ud = un * dist * fx + cx
    vd = vn * dist * fy + cy
    ud = jnp.clip(ud, 0.0, float(IMG_W - 1))
    vd = jnp.clip(vd, 0.0, float(IMG_H - 1))
    idx_ref[0] = (vd / 2.0).astype(jnp.int32) * (IMG_W // 2) + (ud / 2.0).astype(jnp.int32)


def _compute_idx(coef):
    return pl.pallas_call(
        _idx_body,
        grid=(B * C, N_CHUNKS),
        in_specs=[
            pl.BlockSpec((B * C, 24), lambda i, j: (0, 0), memory_space=pltpu.SMEM),
        ],
        out_specs=pl.BlockSpec((1, BLK_ROWS, 128), lambda i, j: (i, j, 0)),
        out_shape=jax.ShapeDtypeStruct((B * C, IDX_ROWS, 128), jnp.int32),
    )(coef)


def _tr_table_body(src_ref, dst_ref):
    dst_ref[0] = src_ref[0].T


def _transpose_table(hm3):
    return pl.pallas_call(
        _tr_table_body,
        grid=(B * C, (IMG_H // 2) * (IMG_W // 2) // 512),
        in_specs=[pl.BlockSpec((1, J, 512), lambda i, j: (i, 0, j))],
        out_specs=pl.BlockSpec((1, 512, J), lambda i, j: (i, j, 0)),
        out_shape=jax.ShapeDtypeStruct((B * C, HW, J), jnp.float32),
    )(hm3)


def _transpose_out(outp3):
    return pl.pallas_call(
        _tr_table_body,
        grid=(B, G3 // 512),
        in_specs=[pl.BlockSpec((1, 512, J), lambda i, j: (i, j, 0))],
        out_specs=pl.BlockSpec((1, J, 512), lambda i, j: (i, 0, j)),
        out_shape=jax.ShapeDtypeStruct((B, J, G3), jnp.float32),
    )(outp3)


def _sc_body(idx_hbm, table_hbm, out_hbm, idx_v, rows_v, urows, outbuf, isem, gsem, usem):
    w = lax.axis_index("s") * 2 + lax.axis_index("c")

    def idx_src(si, c):
        row0 = w * RPW + si * SUP
        b = row0 // G3
        pnt0 = row0 - b * G3
        base = pl.multiple_of(((b * C + c) * G3 + pnt0) // 128, SUPROWS)
        return idx_hbm.at[pl.ds(base, SUPROWS)]

    for c in range(C):
        pltpu.async_copy(idx_src(0, c), idx_v.at[0, c], isem)

    def sup_body(si, carry):
        row0 = pl.multiple_of(w * RPW + si * SUP, SUP)
        slot = lax.rem(si, 2)
        nslot = 1 - slot
        for c in range(C):
            pltpu.make_async_copy(idx_src(si, c), idx_v.at[slot, c], isem).wait()

        @pl.when(si + 1 < NSUP)
        def _():
            for c in range(C):
                pltpu.async_copy(idx_src(si + 1, c), idx_v.at[nslot, c], isem)

        b = row0 // G3
        lanes = lax.iota(jnp.int32, 16)
        unis = []
        udescs = []
        for c in range(C):
            v0 = idx_v[slot, c, 0, pl.ds(0, 16)]

            def chk(t, mv):
                v = idx_v[slot, c, t >> 3, pl.ds((t & 7) * 16, 16)]
                return (jnp.minimum(mv[0], v), jnp.maximum(mv[1], v))

            mnv, mxv = lax.fori_loop(1, SUP // 16, chk, (v0, v0))
            mn = jnp.min(mnv)
            mx = jnp.max(mxv)
            unis.append(mn == mx)
            jidx = ((b * C + c) * J + lanes) * HW + mn
            udescs.append(pltpu.async_copy(table_hbm.at[jidx], urows.at[c, 0], usem))
        for d in udescs:
            d.wait()
        all_uni = unis[0]
        for c in range(1, C):
            all_uni = all_uni & unis[c]

        @pl.when(all_uni)
        def _():
            acc = urows[0, 0]
            for c in range(1, C):
                acc = acc + urows[c, 0]
            accv = acc * (1.0 / C)
            for j in range(J):
                sj = jnp.max(jnp.where(lanes == j, accv, jnp.float32(-3.4e38)))
                splat = jnp.full((16,), sj, jnp.float32)

                @plsc.parallel_loop(0, SUP // 16, unroll=8)
                def _(t, j=j, splat=splat):
                    outbuf[j, pl.ds(t * 16, 16)] = splat

        @pl.when(jnp.logical_not(all_uni))
        def _():
            for c in range(C):
                @pl.when(unis[c])
                def _(c=c):
                    row = urows[c, 0]

                    @plsc.parallel_loop(0, J * (SUB // 16), unroll=4)
                    def _(t):
                        j = t >> 3
                        g = t & 7
                        sj = jnp.max(jnp.where(lanes == j, row, jnp.float32(-3.4e38)))
                        rows_v[c, j, pl.ds(g * 16, 16)] = jnp.full((16,), sj, jnp.float32)

            for s in range(NSUB):
                for c in range(C):
                    @pl.when(jnp.logical_not(unis[c]))
                    def _(c=c, s=s):
                        def gather_j(j, _):
                            base = ((b * C + c) * J + j) * HW
                            descs = [
                                pltpu.async_copy(
                                    table_hbm.at[
                                        idx_v[slot, c, s, pl.ds(g * 16, 16)] + base
                                    ],
                                    rows_v.at[c, j, pl.ds(g * 16, 16)],
                                    gsem,
                                )
                                for g in range(8)
                            ]
                            for d in descs:
                                d.wait()
                            return 0

                        lax.fori_loop(0, J, gather_j, 0)

                @plsc.parallel_loop(0, J * (SUB // 16), unroll=4)
                def _(t, s=s):
                    j = t >> 3
                    g = t & 7
                    acc = rows_v[0, j, pl.ds(g * 16, 16)]
                    for c in range(1, C):
                        acc = acc + rows_v[c, j, pl.ds(g * 16, 16)]
                    outbuf[j, pl.ds(s * SUB + g * 16, 16)] = acc * (1.0 / C)

        b_out = row0 // G3
        pnt0_out = row0 - b_out * G3
        pltpu.sync_copy(outbuf, out_hbm.at[b_out, :, pl.ds(pnt0_out, SUP)])
        return carry

    lax.fori_loop(0, NSUP, sup_body, 0)


@functools.cache
def _sc_gather():
    return pl.kernel(
        _sc_body,
        out_type=jax.ShapeDtypeStruct((B, J, G3), jnp.float32),
        mesh=plsc.VectorSubcoreMesh(
            core_axis_name="c", subcore_axis_name="s", num_cores=2, num_subcores=16
        ),
        scratch_types=[
            pltpu.VMEM((2, C, SUPROWS, 128), jnp.int32),
            pltpu.VMEM((C, J, SUB), jnp.float32),
            pltpu.VMEM((C, 1, J), jnp.float32),
            pltpu.VMEM((J, SUP), jnp.float32),
            pltpu.SemaphoreType.DMA,
            pltpu.SemaphoreType.DMA,
            pltpu.SemaphoreType.DMA,
        ],
        compiler_params=pltpu.CompilerParams(
            use_tc_tiling_on_sc=False,
            disable_bounds_checks=True,
            disable_semaphore_checks=True,
            needs_layout_passes=False,
        ),
    )


def kernel(heatmaps, center, cameraMatrices, intrinsicMatrices, distortionCoefficients):
    Bv, Cv, Jv, H, W = heatmaps.shape
    Mf = cameraMatrices.reshape(B * C, 12)
    cen = jnp.broadcast_to(center[:, None, :], (B, C, 3)).reshape(B * C, 3)
    fx = intrinsicMatrices[:, :, 0, 0].reshape(-1, 1)
    fy = intrinsicMatrices[:, :, 1, 1].reshape(-1, 1)
    cx = intrinsicMatrices[:, :, 2, 0].reshape(-1, 1)
    cy = intrinsicMatrices[:, :, 2, 1].reshape(-1, 1)
    k1 = distortionCoefficients[:, :, 0, 0].reshape(-1, 1)
    k2 = distortionCoefficients[:, :, 0, 1].reshape(-1, 1)
    pad = jnp.zeros((B * C, 3), jnp.float32)
    coef = jnp.concatenate([Mf, cen, fx, fy, cx, cy, k1, k2, pad], axis=1)

    idx = _compute_idx(coef)
    idx2d = idx.reshape(B * C * IDX_ROWS, 128)
    table = heatmaps.reshape(B * C * J * H * W)
    outp = _sc_gather()(idx2d, table)
    return outp.reshape(B, J, G, G, G)
